# baseline (speedup 1.0000x reference)
import jax
import jax.numpy as jnp
from jax import lax
from jax.experimental import pallas as pl
from jax.experimental.pallas import tpu as pltpu
from jax.experimental.pallas import tpu_sc as plsc

B = 4096
L = 200
D = 64
VOCAB = 1000000
NC = 2
NS = 16
NW = NC * NS
BPW = B // NW
LP = 208
CPG = 16 * LP // 128
GPW = BPW // 16
NBUF = 4

VB = 32768


def _scores_body(tt_ref, w_ref, s_ref):
    s_ref[...] = jnp.sum(tt_ref[...] * w_ref[...], axis=0)


def _pool_body(idx_hbm, scores_hbm, bias_hbm, out_hbm,
               idx_v, buf0, buf1, buf2, buf3,
               bias_v, out_v,
               sem0, sem1, sem2, sem3):
    c = lax.axis_index("c")
    s = lax.axis_index("s")
    wid = s * NC + c

    bufs = ((buf0, sem0), (buf1, sem1), (buf2, sem2), (buf3, sem3))

    pltpu.sync_copy(idx_hbm.at[wid], idx_v)
    pltpu.sync_copy(bias_hbm, bias_v)

    bv = bias_v[...]
    lane = lax.iota(jnp.int32, 16)
    zero = jnp.zeros((16,), jnp.float32)
    tail_mask = lane < 8
    inv_l = jnp.float32(1.0 / L)
    dnums = lax.GatherDimensionNumbers(
        offset_dims=(), collapsed_slice_dims=(0,), start_index_map=(0,))

    def fire(g, buf, sem):
        for k in range(CPG):
            pltpu.async_copy(scores_hbm.at[idx_v.at[g * CPG + k]],
                             buf.at[pl.ds(k * 128, 128)], sem)

    def drain(buf, sem):
        pltpu.make_async_copy(scores_hbm.at[pl.ds(0, 16 * LP)], buf,
                              sem).wait()

    for p in range(NBUF - 1):
        fire(p, *bufs[p])

    def group4_body(g2, carry):
        for p in range(NBUF):
            g = g2 * NBUF + p
            nxt = g + NBUF - 1
            buf, sem = bufs[p]

            @pl.when(nxt < GPW)
            def _():
                fire(nxt, *bufs[(p + NBUF - 1) % NBUF])

            drain(buf, sem)

            zgroup = zero
            for j in range(16):
                acc = jnp.where(tail_mask, buf[pl.ds(j * LP + 192, 16)], 0.0)
                for i in range(12):
                    acc = acc + buf[pl.ds(j * LP + i * 16, 16)]
                zv = acc * inv_l + bv
                for k in (1, 2, 4, 8):
                    shuf = lax.gather(
                        zv, (lane ^ k)[:, None], dnums, slice_sizes=(1,),
                        mode=lax.GatherScatterMode.PROMISE_IN_BOUNDS)
                    zv = zv + shuf
                zgroup = jnp.where(lane == j, zv, zgroup)

            out_v[pl.ds(g * 16, 16)] = 1.0 / (1.0 + jnp.exp(-zgroup))
        return carry

    lax.fori_loop(0, GPW // NBUF, group4_body, 0)
    pltpu.sync_copy(out_v, out_hbm.at[pl.ds(wid * BPW, BPW)])


def kernel(x, embed_table, fc_w, fc_b):
    wcol = fc_w.reshape(D, 1).astype(jnp.float32)
    scores = pl.pallas_call(
        _scores_body,
        grid=(pl.cdiv(VOCAB, VB),),
        in_specs=[
            pl.BlockSpec((D, VB), lambda i: (0, i)),
            pl.BlockSpec((D, 1), lambda i: (0, 0)),
        ],
        out_specs=pl.BlockSpec((VB,), lambda i: (i,)),
        out_shape=jax.ShapeDtypeStruct((VOCAB,), jnp.float32),
    )(embed_table.T, wcol)

    xpad = jnp.pad(x.astype(jnp.int32), ((0, 0), (0, LP - L)))
    idx3 = xpad.reshape(NW, GPW * CPG, 128)
    bpad = jnp.pad(fc_b.astype(jnp.float32), (0, 15))

    mesh = plsc.VectorSubcoreMesh(core_axis_name="c", subcore_axis_name="s")
    run = pl.kernel(
        _pool_body,
        out_type=jax.ShapeDtypeStruct((B,), jnp.float32),
        mesh=mesh,
        compiler_params=pltpu.CompilerParams(use_tc_tiling_on_sc=False),
        scratch_types=(
            [pltpu.VMEM((GPW * CPG, 128), jnp.int32)]
            + [pltpu.VMEM((16 * LP,), jnp.float32) for _ in range(NBUF)]
            + [pltpu.VMEM((16,), jnp.float32), pltpu.VMEM((BPW,), jnp.float32)]
            + [pltpu.SemaphoreType.DMA for _ in range(NBUF)]
        ),
    )
    out = run(idx3, scores, bpad)
    return out.reshape(B, 1)
